# baseline (speedup 1.0000x reference)
import functools

import jax
import jax.numpy as jnp
from jax import lax
from jax.experimental import pallas as pl
from jax.experimental.pallas import tpu as pltpu
from jax.experimental.pallas import tpu_sc as plsc

NC = 2
NS = 16
DEGW = 16


def _sc_mesh():
    return plsc.VectorSubcoreMesh(
        core_axis_name="c", subcore_axis_name="s",
        num_cores=NC, num_subcores=NS)


def _pick_chunk(per_tile, cap=128):
    for c in range(cap, 0, -8):
        if per_tile % c == 0:
            return c
    return None


def _pad_nodes(N):
    per_tile = -(-N // (NS * 8)) * 8
    return NS * per_tile, per_tile


def _segsum(N, W, E):
    e_per_tile = E // NS
    chunk = _pick_chunk(e_per_tile)
    n_iters = e_per_tile // chunk
    n_pad, n_per_tile = _pad_nodes(N)

    assert n_iters % 2 == 1
    groups = chunk // 16

    @functools.partial(
        pl.kernel,
        out_type=jax.ShapeDtypeStruct((NC, n_pad, W), jnp.float32),
        mesh=_sc_mesh(),
        scratch_types=[
            pltpu.VMEM((e_per_tile,), jnp.int32),
            pltpu.VMEM((e_per_tile,), jnp.int32),
            pltpu.VMEM((chunk, W), jnp.float32),
            pltpu.VMEM((chunk, W), jnp.float32),
            pltpu.VMEM_SHARED((n_pad, W), jnp.float32),
            pltpu.SemaphoreType.DMA,
            pltpu.SemaphoreType.DMA,
            pltpu.SemaphoreType.DMA,
            pltpu.SemaphoreType.DMA,
        ])
    def k(x_hbm, src_hbm, dst_hbm, zeros_hbm, agg_out,
          sidx, didx, rows_a, rows_b, agg_sh, sem_a, sem_b, sem_sa,
          sem_sb):
        c = lax.axis_index("c")
        s = lax.axis_index("s")
        rbase = s * n_per_tile
        pltpu.sync_copy(zeros_hbm.at[pl.ds(rbase, n_per_tile)],
                        agg_sh.at[pl.ds(rbase, n_per_tile)])
        e_base = s * e_per_tile
        pltpu.sync_copy(src_hbm.at[pl.ds(e_base, e_per_tile)], sidx)
        pltpu.sync_copy(dst_hbm.at[pl.ds(e_base, e_per_tile)], didx)
        coff = c

        def offs(g, _):
            sl = pl.ds(g * 16, 16)
            sidx[sl] = sidx[sl] * NC + coff
            return 0

        lax.fori_loop(0, e_per_tile // 16, offs, 0)
        plsc.subcore_barrier()

        def gather(i, buf, sem):
            pltpu.async_copy(
                x_hbm.at[sidx.at[pl.ds(i * chunk, chunk)]], buf, sem)

        def gwait(buf, sem):
            pltpu.make_async_copy(zeros_hbm.at[pl.ds(0, chunk)], buf,
                                  sem).wait()

        def sfire(i, buf, sem):
            for j in range(groups):
                idx_v = didx[pl.ds(i * chunk + j * 16, 16)]
                pltpu.async_copy(buf.at[pl.ds(j * 16, 16)],
                                 agg_sh.at[idx_v], sem, add=True)

        def sdrain(buf, sem):
            for j in range(groups):
                pltpu.make_async_copy(zeros_hbm.at[pl.ds(0, 16)],
                                      buf.at[pl.ds(j * 16, 16)], sem).wait()

        gather(0, rows_a, sem_a)

        def body(kk, _):
            i0 = 2 * kk
            gather(i0 + 1, rows_b, sem_b)
            gwait(rows_a, sem_a)
            sfire(i0, rows_a, sem_sa)
            gwait(rows_b, sem_b)
            sfire(i0 + 1, rows_b, sem_sb)
            sdrain(rows_a, sem_sa)
            gather(i0 + 2, rows_a, sem_a)
            sdrain(rows_b, sem_sb)
            return 0

        lax.fori_loop(0, (n_iters - 1) // 2, body, 0)
        gwait(rows_a, sem_a)
        sfire(n_iters - 1, rows_a, sem_sa)
        sdrain(rows_a, sem_sa)
        plsc.subcore_barrier()
        pltpu.sync_copy(agg_sh.at[pl.ds(rbase, n_per_tile)],
                        agg_out.at[c].at[pl.ds(rbase, n_per_tile)])

    return k


def _deg(N, E, W):
    e_per_tile = E // NS
    chunk = _pick_chunk(e_per_tile)
    n_iters = e_per_tile // chunk
    n_pad, n_per_tile = _pad_nodes(N)

    c0_chunks = (n_iters + 1) // 2
    groups = chunk // 16

    @functools.partial(
        pl.kernel,
        out_type=jax.ShapeDtypeStruct((NC, n_pad, W), jnp.float32),
        mesh=_sc_mesh(),
        scratch_types=[
            pltpu.VMEM((e_per_tile,), jnp.int32),
            pltpu.VMEM((16, W), jnp.float32),
            pltpu.VMEM_SHARED((n_pad, W), jnp.float32),
            pltpu.SemaphoreType.DMA,
        ])
    def k(dst_hbm, zeros_hbm, ones_hbm, deg_out, didx, ones_v, deg_sh,
          sem_s):
        c = lax.axis_index("c")
        s = lax.axis_index("s")
        rbase = s * n_per_tile
        pltpu.sync_copy(zeros_hbm.at[pl.ds(rbase, n_per_tile)],
                        deg_sh.at[pl.ds(rbase, n_per_tile)])
        pltpu.sync_copy(ones_hbm, ones_v)
        e_base = s * e_per_tile
        pltpu.sync_copy(dst_hbm.at[pl.ds(e_base, e_per_tile)], didx)
        plsc.subcore_barrier()

        cstart = c * c0_chunks
        n_chunks = c0_chunks - c

        def body(i, _):
            base = (cstart + i) * chunk
            descs = []
            for j in range(groups):
                idx_v = didx[pl.ds(base + j * 16, 16)]
                descs.append(pltpu.async_copy(
                    ones_v, deg_sh.at[idx_v], sem_s, add=True))
            for d in descs:
                d.wait()
            return 0

        lax.fori_loop(0, n_chunks, body, 0)
        plsc.subcore_barrier()
        pltpu.sync_copy(deg_sh.at[pl.ds(rbase, n_per_tile)],
                        deg_out.at[c].at[pl.ds(rbase, n_per_tile)])

    return k


def _gather2(N, H, Q):
    q_per_tile = Q // NS
    chunk = _pick_chunk(q_per_tile)
    n_iters = q_per_tile // chunk

    assert n_iters % 2 == 0

    @functools.partial(
        pl.kernel,
        out_type=[jax.ShapeDtypeStruct((NC, Q, H), jnp.float32),
                  jax.ShapeDtypeStruct((NC, Q, H), jnp.float32)],
        mesh=_sc_mesh(),
        scratch_types=[
            pltpu.VMEM((q_per_tile,), jnp.int32),
            pltpu.VMEM((q_per_tile,), jnp.int32),
            pltpu.VMEM((chunk, H), jnp.float32),
            pltpu.VMEM((chunk, H), jnp.float32),
            pltpu.VMEM((chunk, H), jnp.float32),
            pltpu.VMEM((chunk, H), jnp.float32),
            pltpu.SemaphoreType.DMA,
            pltpu.SemaphoreType.DMA,
            pltpu.SemaphoreType.DMA,
            pltpu.SemaphoreType.DMA,
        ])
    def k(x_hbm, e0_hbm, e1_hbm, g0_out, g1_out, idx0, idx1,
          a0, a1, b0, b1, sem_a, sem_b, sem_wa, sem_wb):
        c = lax.axis_index("c")
        s = lax.axis_index("s")
        coff = c
        qbase = s * q_per_tile
        pltpu.sync_copy(e0_hbm.at[pl.ds(qbase, q_per_tile)], idx0)
        pltpu.sync_copy(e1_hbm.at[pl.ds(qbase, q_per_tile)], idx1)

        def offs(g, _):
            sl = pl.ds(g * 16, 16)
            idx0[sl] = idx0[sl] * NC + coff
            idx1[sl] = idx1[sl] * NC + coff
            return 0

        lax.fori_loop(0, q_per_tile // 16, offs, 0)

        def gathers(i, r0, r1, sem):
            sl = pl.ds(i * chunk, chunk)
            pltpu.async_copy(x_hbm.at[idx0.at[sl]], r0, sem)
            pltpu.async_copy(x_hbm.at[idx1.at[sl]], r1, sem)

        def gwait(r0, r1, sem):
            pltpu.make_async_copy(x_hbm.at[pl.ds(0, chunk)], r0, sem).wait()
            pltpu.make_async_copy(x_hbm.at[pl.ds(0, chunk)], r1, sem).wait()

        def writes(i, r0, r1, sem):
            sl = pl.ds(qbase + i * chunk, chunk)
            pltpu.async_copy(r0, g0_out.at[c].at[sl], sem)
            pltpu.async_copy(r1, g1_out.at[c].at[sl], sem)

        def wdrain(r0, r1, sem):
            pltpu.make_async_copy(x_hbm.at[pl.ds(0, chunk)], r0, sem).wait()
            pltpu.make_async_copy(x_hbm.at[pl.ds(0, chunk)], r1, sem).wait()

        gathers(0, a0, a1, sem_a)

        def body(kk, _):
            i0 = 2 * kk
            gathers(i0 + 1, b0, b1, sem_b)
            gwait(a0, a1, sem_a)
            writes(i0, a0, a1, sem_wa)
            gwait(b0, b1, sem_b)
            writes(i0 + 1, b0, b1, sem_wb)
            wdrain(a0, a1, sem_wa)
            gathers(i0 + 2, a0, a1, sem_a)
            wdrain(b0, b1, sem_wb)
            return 0

        lax.fori_loop(0, n_iters // 2 - 1, body, 0)
        gathers(n_iters - 1, b0, b1, sem_b)
        gwait(a0, a1, sem_a)
        writes(n_iters - 2, a0, a1, sem_wa)
        gwait(b0, b1, sem_b)
        writes(n_iters - 1, b0, b1, sem_wb)
        wdrain(a0, a1, sem_wa)
        wdrain(b0, b1, sem_wb)

    return k


def _layer_tc(N, D, H, relu, wagg):
    R = 2000

    def body(agg_ref, deg_ref, x_ref, wl_ref, bl_ref, wr_ref, out_ref):
        agg = jnp.concatenate(
            [agg_ref[0][:, :H], agg_ref[1][:, :H]], axis=1)
        x = x_ref[...].reshape(R, 2 * H)
        d = jnp.maximum(deg_ref[0][:, 0:1] + deg_ref[1][:, 0:1], 1.0)
        res = (jnp.dot(agg / d, wl_ref[...], preferred_element_type=jnp.float32)
               + bl_ref[...]
               + jnp.dot(x, wr_ref[...], preferred_element_type=jnp.float32))
        if relu:
            res = jnp.maximum(res, 0.0)
        out_ref[...] = res.reshape(R, NC, H)

    return pl.pallas_call(
        body,
        grid=(N // R,),
        in_specs=[
            pl.BlockSpec((2, R, wagg), lambda i: (0, i, 0)),
            pl.BlockSpec((2, R, H), lambda i: (0, i, 0)),
            pl.BlockSpec((R, NC, H), lambda i: (i, 0, 0)),
            pl.BlockSpec((D, D), lambda i: (0, 0)),
            pl.BlockSpec((1, D), lambda i: (0, 0)),
            pl.BlockSpec((D, D), lambda i: (0, 0)),
        ],
        out_specs=pl.BlockSpec((R, NC, H), lambda i: (i, 0, 0)),
        out_shape=jax.ShapeDtypeStruct((N, NC, H), jnp.float32),
    )


def _predict_tc(Q, D, H):
    R = 4096

    def body(g0_ref, g1_ref, w1_ref, b1_ref, w2_ref, b2_ref, out_ref):
        a = jnp.concatenate([g0_ref[0], g0_ref[1]], axis=1)
        b = jnp.concatenate([g1_ref[0], g1_ref[1]], axis=1)
        h = a * b
        h = jnp.maximum(
            jnp.dot(h, w1_ref[...], preferred_element_type=jnp.float32)
            + b1_ref[...], 0.0)
        o = (jnp.dot(h, w2_ref[...], preferred_element_type=jnp.float32)
             + b2_ref[...])
        out_ref[...] = jax.nn.sigmoid(o)

    return pl.pallas_call(
        body,
        grid=(Q // R,),
        in_specs=[
            pl.BlockSpec((2, R, H), lambda i: (0, i, 0)),
            pl.BlockSpec((2, R, H), lambda i: (0, i, 0)),
            pl.BlockSpec((D, D), lambda i: (0, 0)),
            pl.BlockSpec((1, D), lambda i: (0, 0)),
            pl.BlockSpec((D, 1), lambda i: (0, 0)),
            pl.BlockSpec((1, 1), lambda i: (0, 0)),
        ],
        out_specs=pl.BlockSpec((R, 1), lambda i: (i, 0)),
        out_shape=jax.ShapeDtypeStruct((Q, 1), jnp.float32),
    )


def kernel(adj_t, edges, emb, Wl1, bl1, Wr1, Wl2, bl2, Wr2, Wp1, bp1, Wp2, bp2):
    N, D = emb.shape
    E = adj_t.shape[1]
    Q = edges.shape[1]
    H = D // NC

    segsum = _segsum(N, H, E)
    deg_k = _deg(N, E, H)
    gather2 = _gather2(N, H, Q)
    layer1 = _layer_tc(N, D, H, relu=True, wagg=H)
    layer2 = _layer_tc(N, D, H, relu=False, wagg=H)
    predict = _predict_tc(Q, D, H)

    n_pad, _ = _pad_nodes(N)
    zeros = jnp.zeros((n_pad, H), jnp.float32)
    ones16 = jnp.ones((16, H), jnp.float32)

    xs0 = emb.reshape(N, NC, H)
    e_src, e_dst = adj_t[0], adj_t[1]
    dega = deg_k(e_dst, zeros, ones16)
    agg1 = segsum(xs0.reshape(NC * N, H), e_src, e_dst, zeros)
    xs1 = layer1(agg1, dega, xs0, Wl1, bl1.reshape(1, D), Wr1)
    agg2 = segsum(xs1.reshape(NC * N, H), e_src, e_dst, zeros)
    xs2 = layer2(agg2, dega, xs1, Wl2, bl2.reshape(1, D), Wr2)
    g0, g1 = gather2(xs2.reshape(NC * N, H), edges[0], edges[1])
    out = predict(g0, g1, Wp1, bp1.reshape(1, D), Wp2, bp2.reshape(1, 1))
    return out


# speedup vs baseline: 1.1297x; 1.1297x over previous
import functools

import jax
import jax.numpy as jnp
from jax import lax
from jax.experimental import pallas as pl
from jax.experimental.pallas import tpu as pltpu
from jax.experimental.pallas import tpu_sc as plsc

NC = 2
NS = 16
DEGW = 16


def _sc_mesh():
    return plsc.VectorSubcoreMesh(
        core_axis_name="c", subcore_axis_name="s",
        num_cores=NC, num_subcores=NS)


def _pick_chunk(per_tile, cap=128):
    for c in range(cap, 0, -8):
        if per_tile % c == 0:
            return c
    return None


def _pad_nodes(N):
    per_tile = -(-N // (NS * 8)) * 8
    return NS * per_tile, per_tile


def _segsum(N, W, E):
    e_per_tile = E // NS
    chunk = _pick_chunk(e_per_tile)
    n_iters = e_per_tile // chunk
    n_pad, n_per_tile = _pad_nodes(N)

    assert n_iters % 2 == 1
    groups = chunk // 16

    @functools.partial(
        pl.kernel,
        out_type=jax.ShapeDtypeStruct((NC, n_pad, W), jnp.float32),
        mesh=_sc_mesh(),
        scratch_types=[
            pltpu.VMEM((e_per_tile,), jnp.int32),
            pltpu.VMEM((e_per_tile,), jnp.int32),
            pltpu.VMEM((chunk, W), jnp.float32),
            pltpu.VMEM((chunk, W), jnp.float32),
            pltpu.VMEM_SHARED((n_pad, W), jnp.float32),
            pltpu.SemaphoreType.DMA,
            pltpu.SemaphoreType.DMA,
            pltpu.SemaphoreType.DMA,
            pltpu.SemaphoreType.DMA,
        ])
    def k(x_hbm, src_hbm, dst_hbm, zeros_hbm, agg_out,
          sidx, didx, rows_a, rows_b, agg_sh, sem_a, sem_b, sem_sa,
          sem_sb):
        c = lax.axis_index("c")
        s = lax.axis_index("s")
        rbase = s * n_per_tile
        pltpu.sync_copy(zeros_hbm.at[pl.ds(rbase, n_per_tile)],
                        agg_sh.at[pl.ds(rbase, n_per_tile)])
        e_base = s * e_per_tile
        pltpu.sync_copy(src_hbm.at[pl.ds(e_base, e_per_tile)], sidx)
        pltpu.sync_copy(dst_hbm.at[pl.ds(e_base, e_per_tile)], didx)
        coff = c

        def offs(g, _):
            sl = pl.ds(g * 16, 16)
            sidx[sl] = sidx[sl] * NC + coff
            return 0

        lax.fori_loop(0, e_per_tile // 16, offs, 0)
        plsc.subcore_barrier()

        def gather(i, buf, sem):
            pltpu.async_copy(
                x_hbm.at[sidx.at[pl.ds(i * chunk, chunk)]], buf, sem)

        def gwait(buf, sem):
            pltpu.make_async_copy(zeros_hbm.at[pl.ds(0, chunk)], buf,
                                  sem).wait()

        def scatter(i, buf, sem):
            descs = []
            for j in range(groups):
                idx_v = didx[pl.ds(i * chunk + j * 16, 16)]
                descs.append(pltpu.async_copy(
                    buf.at[pl.ds(j * 16, 16)], agg_sh.at[idx_v], sem,
                    add=True))
            for d in descs:
                d.wait()

        gather(0, rows_a, sem_a)

        def body(kk, _):
            i0 = 2 * kk
            gather(i0 + 1, rows_b, sem_b)
            gwait(rows_a, sem_a)
            scatter(i0, rows_a, sem_sa)
            gather(i0 + 2, rows_a, sem_a)
            gwait(rows_b, sem_b)
            scatter(i0 + 1, rows_b, sem_sb)
            return 0

        lax.fori_loop(0, (n_iters - 1) // 2, body, 0)
        gwait(rows_a, sem_a)
        scatter(n_iters - 1, rows_a, sem_sa)
        plsc.subcore_barrier()
        pltpu.sync_copy(agg_sh.at[pl.ds(rbase, n_per_tile)],
                        agg_out.at[c].at[pl.ds(rbase, n_per_tile)])

    return k


def _deg(N, E, W):
    e_per_tile = E // NS
    chunk = _pick_chunk(e_per_tile)
    n_iters = e_per_tile // chunk
    n_pad, n_per_tile = _pad_nodes(N)

    c0_chunks = (n_iters + 1) // 2
    groups = chunk // 16

    @functools.partial(
        pl.kernel,
        out_type=jax.ShapeDtypeStruct((NC, n_pad, W), jnp.float32),
        mesh=_sc_mesh(),
        scratch_types=[
            pltpu.VMEM((e_per_tile,), jnp.int32),
            pltpu.VMEM((16, W), jnp.float32),
            pltpu.VMEM_SHARED((n_pad, W), jnp.float32),
            pltpu.SemaphoreType.DMA,
        ])
    def k(dst_hbm, zeros_hbm, ones_hbm, deg_out, didx, ones_v, deg_sh,
          sem_s):
        c = lax.axis_index("c")
        s = lax.axis_index("s")
        rbase = s * n_per_tile
        pltpu.sync_copy(zeros_hbm.at[pl.ds(rbase, n_per_tile)],
                        deg_sh.at[pl.ds(rbase, n_per_tile)])
        pltpu.sync_copy(ones_hbm, ones_v)
        e_base = s * e_per_tile
        pltpu.sync_copy(dst_hbm.at[pl.ds(e_base, e_per_tile)], didx)
        plsc.subcore_barrier()

        cstart = c * c0_chunks
        n_chunks = c0_chunks - c

        def body(i, _):
            base = (cstart + i) * chunk
            descs = []
            for j in range(groups):
                idx_v = didx[pl.ds(base + j * 16, 16)]
                descs.append(pltpu.async_copy(
                    ones_v, deg_sh.at[idx_v], sem_s, add=True))
            for d in descs:
                d.wait()
            return 0

        lax.fori_loop(0, n_chunks, body, 0)
        plsc.subcore_barrier()
        pltpu.sync_copy(deg_sh.at[pl.ds(rbase, n_per_tile)],
                        deg_out.at[c].at[pl.ds(rbase, n_per_tile)])

    return k


def _gather2(N, H, Q):
    q_per_tile = Q // NS
    chunk = _pick_chunk(q_per_tile)
    n_iters = q_per_tile // chunk

    assert n_iters % 2 == 0

    @functools.partial(
        pl.kernel,
        out_type=[jax.ShapeDtypeStruct((NC, Q, H), jnp.float32),
                  jax.ShapeDtypeStruct((NC, Q, H), jnp.float32)],
        mesh=_sc_mesh(),
        scratch_types=[
            pltpu.VMEM((q_per_tile,), jnp.int32),
            pltpu.VMEM((q_per_tile,), jnp.int32),
            pltpu.VMEM((chunk, H), jnp.float32),
            pltpu.VMEM((chunk, H), jnp.float32),
            pltpu.VMEM((chunk, H), jnp.float32),
            pltpu.VMEM((chunk, H), jnp.float32),
            pltpu.SemaphoreType.DMA,
            pltpu.SemaphoreType.DMA,
            pltpu.SemaphoreType.DMA,
            pltpu.SemaphoreType.DMA,
        ])
    def k(x_hbm, e0_hbm, e1_hbm, g0_out, g1_out, idx0, idx1,
          a0, a1, b0, b1, sem_a, sem_b, sem_wa, sem_wb):
        c = lax.axis_index("c")
        s = lax.axis_index("s")
        coff = c
        qbase = s * q_per_tile
        pltpu.sync_copy(e0_hbm.at[pl.ds(qbase, q_per_tile)], idx0)
        pltpu.sync_copy(e1_hbm.at[pl.ds(qbase, q_per_tile)], idx1)

        def offs(g, _):
            sl = pl.ds(g * 16, 16)
            idx0[sl] = idx0[sl] * NC + coff
            idx1[sl] = idx1[sl] * NC + coff
            return 0

        lax.fori_loop(0, q_per_tile // 16, offs, 0)

        def gathers(i, r0, r1, sem):
            sl = pl.ds(i * chunk, chunk)
            pltpu.async_copy(x_hbm.at[idx0.at[sl]], r0, sem)
            pltpu.async_copy(x_hbm.at[idx1.at[sl]], r1, sem)

        def gwait(r0, r1, sem):
            pltpu.make_async_copy(x_hbm.at[pl.ds(0, chunk)], r0, sem).wait()
            pltpu.make_async_copy(x_hbm.at[pl.ds(0, chunk)], r1, sem).wait()

        def writes(i, r0, r1, sem):
            sl = pl.ds(qbase + i * chunk, chunk)
            pltpu.async_copy(r0, g0_out.at[c].at[sl], sem)
            pltpu.async_copy(r1, g1_out.at[c].at[sl], sem)

        def wdrain(r0, r1, sem):
            pltpu.make_async_copy(x_hbm.at[pl.ds(0, chunk)], r0, sem).wait()
            pltpu.make_async_copy(x_hbm.at[pl.ds(0, chunk)], r1, sem).wait()

        gathers(0, a0, a1, sem_a)

        def body(kk, _):
            i0 = 2 * kk
            gathers(i0 + 1, b0, b1, sem_b)
            gwait(a0, a1, sem_a)
            writes(i0, a0, a1, sem_wa)
            gwait(b0, b1, sem_b)
            writes(i0 + 1, b0, b1, sem_wb)
            wdrain(a0, a1, sem_wa)
            gathers(i0 + 2, a0, a1, sem_a)
            wdrain(b0, b1, sem_wb)
            return 0

        lax.fori_loop(0, n_iters // 2 - 1, body, 0)
        gathers(n_iters - 1, b0, b1, sem_b)
        gwait(a0, a1, sem_a)
        writes(n_iters - 2, a0, a1, sem_wa)
        gwait(b0, b1, sem_b)
        writes(n_iters - 1, b0, b1, sem_wb)
        wdrain(a0, a1, sem_wa)
        wdrain(b0, b1, sem_wb)

    return k


def _layer_tc(N, D, H, relu, wagg):
    R = 2000

    def body(agg_ref, deg_ref, x_ref, wl_ref, bl_ref, wr_ref, out_ref):
        agg = jnp.concatenate(
            [agg_ref[0][:, :H], agg_ref[1][:, :H]], axis=1)
        x = x_ref[...].reshape(R, 2 * H)
        d = jnp.maximum(deg_ref[0][:, 0:1] + deg_ref[1][:, 0:1], 1.0)
        res = (jnp.dot(agg / d, wl_ref[...], preferred_element_type=jnp.float32)
               + bl_ref[...]
               + jnp.dot(x, wr_ref[...], preferred_element_type=jnp.float32))
        if relu:
            res = jnp.maximum(res, 0.0)
        out_ref[...] = res.reshape(R, NC, H)

    return pl.pallas_call(
        body,
        grid=(N // R,),
        in_specs=[
            pl.BlockSpec((2, R, wagg), lambda i: (0, i, 0)),
            pl.BlockSpec((2, R, H), lambda i: (0, i, 0)),
            pl.BlockSpec((R, NC, H), lambda i: (i, 0, 0)),
            pl.BlockSpec((D, D), lambda i: (0, 0)),
            pl.BlockSpec((1, D), lambda i: (0, 0)),
            pl.BlockSpec((D, D), lambda i: (0, 0)),
        ],
        out_specs=pl.BlockSpec((R, NC, H), lambda i: (i, 0, 0)),
        out_shape=jax.ShapeDtypeStruct((N, NC, H), jnp.float32),
    )


def _predict_tc(Q, D, H):
    R = 4096

    def body(g0_ref, g1_ref, w1_ref, b1_ref, w2_ref, b2_ref, out_ref):
        a = jnp.concatenate([g0_ref[0], g0_ref[1]], axis=1)
        b = jnp.concatenate([g1_ref[0], g1_ref[1]], axis=1)
        h = a * b
        h = jnp.maximum(
            jnp.dot(h, w1_ref[...], preferred_element_type=jnp.float32)
            + b1_ref[...], 0.0)
        o = (jnp.dot(h, w2_ref[...], preferred_element_type=jnp.float32)
             + b2_ref[...])
        out_ref[...] = jax.nn.sigmoid(o)

    return pl.pallas_call(
        body,
        grid=(Q // R,),
        in_specs=[
            pl.BlockSpec((2, R, H), lambda i: (0, i, 0)),
            pl.BlockSpec((2, R, H), lambda i: (0, i, 0)),
            pl.BlockSpec((D, D), lambda i: (0, 0)),
            pl.BlockSpec((1, D), lambda i: (0, 0)),
            pl.BlockSpec((D, 1), lambda i: (0, 0)),
            pl.BlockSpec((1, 1), lambda i: (0, 0)),
        ],
        out_specs=pl.BlockSpec((R, 1), lambda i: (i, 0)),
        out_shape=jax.ShapeDtypeStruct((Q, 1), jnp.float32),
    )


def kernel(adj_t, edges, emb, Wl1, bl1, Wr1, Wl2, bl2, Wr2, Wp1, bp1, Wp2, bp2):
    N, D = emb.shape
    E = adj_t.shape[1]
    Q = edges.shape[1]
    H = D // NC

    segsum = _segsum(N, H, E)
    deg_k = _deg(N, E, H)
    gather2 = _gather2(N, H, Q)
    layer1 = _layer_tc(N, D, H, relu=True, wagg=H)
    layer2 = _layer_tc(N, D, H, relu=False, wagg=H)
    predict = _predict_tc(Q, D, H)

    n_pad, _ = _pad_nodes(N)
    zeros = jnp.zeros((n_pad, H), jnp.float32)
    ones16 = jnp.ones((16, H), jnp.float32)

    xs0 = emb.reshape(N, NC, H)
    e_src, e_dst = adj_t[0], adj_t[1]
    dega = deg_k(e_dst, zeros, ones16)
    agg1 = segsum(xs0.reshape(NC * N, H), e_src, e_dst, zeros)
    xs1 = layer1(agg1, dega, xs0, Wl1, bl1.reshape(1, D), Wr1)
    agg2 = segsum(xs1.reshape(NC * N, H), e_src, e_dst, zeros)
    xs2 = layer2(agg2, dega, xs1, Wl2, bl2.reshape(1, D), Wr2)
    g0, g1 = gather2(xs2.reshape(NC * N, H), edges[0], edges[1])
    out = predict(g0, g1, Wp1, bp1.reshape(1, D), Wp2, bp2.reshape(1, 1))
    return out
